# baseline (speedup 1.0000x reference)
import functools

import jax
import jax.numpy as jnp
from jax import lax
from jax.experimental import pallas as pl
from jax.experimental.pallas import tpu as pltpu
from jax.experimental.pallas import tpu_sc as plsc

_B, _T, _D = 8, 16, 128
_N = _B * _T
_ROWS_PER_W = 8


def _gather_body(idx_hbm, table_hbm, out_hbm, idx_v, rows_v, sem):
    c = lax.axis_index("c")
    s = lax.axis_index("s")
    w = c * 8 + s

    @pl.when(s < 8)
    def _():
        base = w * _ROWS_PER_W
        pltpu.sync_copy(idx_hbm.at[pl.ds(base, _ROWS_PER_W)], idx_v)
        pltpu.async_copy(table_hbm.at[idx_v], rows_v, sem).wait()
        pltpu.sync_copy(rows_v, out_hbm.at[pl.ds(base, _ROWS_PER_W)])


@functools.cache
def _sc_gather():
    return pl.kernel(
        _gather_body,
        out_type=jax.ShapeDtypeStruct((_N, _D), jnp.float32),
        mesh=plsc.VectorSubcoreMesh(core_axis_name="c", subcore_axis_name="s"),
        scratch_types=[
            pltpu.VMEM((_ROWS_PER_W,), jnp.int32),
            pltpu.VMEM((_ROWS_PER_W, _D), jnp.float32),
            pltpu.SemaphoreType.DMA,
        ],
    )


def _loss_body(logits_ref, tgt_ref, loss_ref):
    x = logits_ref[...]
    m = jnp.max(x, axis=1, keepdims=True)
    e = jnp.exp(x - m)
    lse = jnp.log(jnp.sum(e, axis=1, keepdims=True)) + m
    t = tgt_ref[...]
    x3 = x.reshape(_B, _T, _D)
    cols = lax.broadcasted_iota(jnp.int32, (_B, _T, _D), 2)
    picked = jnp.sum(jnp.where(cols == t[:, :, None], x3, 0.0), axis=2)
    loss_ref[0, 0] = (jnp.sum(lse) - jnp.sum(picked)) / _N


def _tc_loss(logits, targets):
    return pl.pallas_call(
        _loss_body,
        out_shape=jax.ShapeDtypeStruct((1, 1), jnp.float32),
        out_specs=pl.BlockSpec(memory_space=pltpu.SMEM),
    )(logits, targets)


def kernel(idx, targets, embedding_table):
    logits = _sc_gather()(idx.reshape(_N), embedding_table)
    loss = _tc_loss(logits, targets)
    return logits, loss[0, 0]
